# baseline (speedup 1.0000x reference)
import dataclasses

import jax
import jax.numpy as jnp
from jax.experimental import pallas as pl
from jax.experimental.pallas import tpu as pltpu
from jax.experimental.pallas import tpu_sc as plsc

NUM_ACT = 96
SIZE = 257
INV_GRID = 32.0
HALF = 128
TAB_N = NUM_ACT * SIZE

LANES = 16
BH = 112
BW = 128


def kernel(x, coefficients_vect):
    b, c, h, w = x.shape
    rows = b * c
    hb = h // BH
    wb = pl.cdiv(w, BW)
    base = ((jnp.arange(rows, dtype=jnp.int32) % NUM_ACT) * SIZE).astype(jnp.float32)
    base_arr = jnp.broadcast_to(base[:, None], (rows, LANES))
    cb = coefficients_vect.astype(jnp.bfloat16)
    nxt = jnp.concatenate([coefficients_vect[1:], coefficients_vect[-1:]])
    sb = (nxt - cb.astype(jnp.float32)).astype(jnp.bfloat16)
    hi16 = jax.lax.bitcast_convert_type(cb, jnp.uint16).astype(jnp.uint32) << 16
    lo16 = jax.lax.bitcast_convert_type(sb, jnp.uint16).astype(jnp.uint32)
    packed_tab = jax.lax.bitcast_convert_type(hi16 | lo16, jnp.int32)

    mesh = plsc.VectorSubcoreMesh(core_axis_name="core", subcore_axis_name="subcore")
    cp = pltpu.CompilerParams(use_tc_tiling_on_sc=True)
    if "needs_layout_passes" in pltpu.CompilerParams.__dataclass_fields__:
        cp = dataclasses.replace(cp, needs_layout_passes=False)

    @pl.kernel(
        out_type=jax.ShapeDtypeStruct((b, c, h, w), jnp.float32),
        mesh=mesh,
        scratch_types=[pltpu.VMEM((TAB_N,), jnp.int32)],
        compiler_params=cp,
    )
    def spline_kernel(x_hbm, base_hbm, ptab_hbm, o_hbm, tab_v):
        pltpu.sync_copy(ptab_hbm, tab_v)

        def body(x_vmem, base_vmem, o_vmem):
            lo = base_vmem[0, :]
            bw_off = lo + jnp.float32(HALF)
            hi = lo + jnp.float32(2 * HALF - 1)

            @plsc.parallel_loop(0, BH * BW, step=LANES, unroll=4)
            def _(k):
                s = k // BW
                l = k % BW
                xv = x_vmem[0, 0, s, pl.ds(l, LANES)]
                wv = xv * INV_GRID + bw_off
                z = jnp.minimum(jnp.maximum(wv, lo), hi)
                i = z.astype(jnp.int32)
                frac = wv - i.astype(jnp.float32)
                idx = jax.lax.bitcast_convert_type(
                    jnp.minimum(
                        jax.lax.bitcast_convert_type(i, jnp.uint32),
                        jnp.uint32(TAB_N - 1),
                    ),
                    jnp.int32,
                )
                pk = plsc.load_gather(tab_v, [idx])
                c0 = jax.lax.bitcast_convert_type(
                    pk & jnp.int32(-65536), jnp.float32
                )
                s0 = jax.lax.bitcast_convert_type(pk << 16, jnp.float32)
                o_vmem[0, 0, s, pl.ds(l, LANES)] = c0 + frac * s0

        pltpu.emit_pipeline(
            body,
            grid=(rows * hb * wb,),
            in_specs=[
                pl.BlockSpec(
                    (1, 1, BH, BW),
                    lambda j: (
                        j // (NUM_ACT * hb * wb),
                        (j // (hb * wb)) % NUM_ACT,
                        (j % (hb * wb)) // wb,
                        j % wb,
                    ),
                ),
                pl.BlockSpec((1, LANES), lambda j: (j // (hb * wb), 0)),
            ],
            out_specs=[
                pl.BlockSpec(
                    (1, 1, BH, BW),
                    lambda j: (
                        j // (NUM_ACT * hb * wb),
                        (j // (hb * wb)) % NUM_ACT,
                        (j % (hb * wb)) // wb,
                        j % wb,
                    ),
                )
            ],
            core_axis_name=("core", "subcore"),
            dimension_semantics=(pltpu.PARALLEL,),
        )(x_hbm, base_hbm, o_hbm)

    out = spline_kernel(x, base_arr, packed_tab)
    return out


# speedup vs baseline: 1.1876x; 1.1876x over previous
import dataclasses

import jax
import jax.numpy as jnp
from jax.experimental import pallas as pl
from jax.experimental.pallas import tpu as pltpu
from jax.experimental.pallas import tpu_sc as plsc

NUM_ACT = 96
SIZE = 257
INV_GRID = 32.0
HALF = 128
TAB_N = NUM_ACT * SIZE

LANES = 16
BH = 112
BW = 128


def kernel(x, coefficients_vect):
    b, c, h, w = x.shape
    rows = b * c
    hb = h // BH
    wb = pl.cdiv(w, BW)
    base = ((jnp.arange(rows, dtype=jnp.int32) % NUM_ACT) * SIZE).astype(jnp.float32)
    base_arr = jnp.broadcast_to(base[:, None], (rows, LANES))
    slope_vect = jnp.concatenate(
        [coefficients_vect[1:] - coefficients_vect[:-1],
         jnp.zeros((1,), jnp.float32)]
    )

    mesh = plsc.VectorSubcoreMesh(core_axis_name="core", subcore_axis_name="subcore")
    cp = pltpu.CompilerParams(use_tc_tiling_on_sc=True)
    if "needs_layout_passes" in pltpu.CompilerParams.__dataclass_fields__:
        cp = dataclasses.replace(cp, needs_layout_passes=False)

    @pl.kernel(
        out_type=jax.ShapeDtypeStruct((b, c, h, w), jnp.float32),
        mesh=mesh,
        scratch_types=[
            pltpu.VMEM((TAB_N,), jnp.float32),
            pltpu.VMEM((TAB_N,), jnp.float32),
        ],
        compiler_params=cp,
    )
    def spline_kernel(x_hbm, base_hbm, coeff_hbm, slope_hbm, o_hbm, tab_v, slp_v):
        pltpu.sync_copy(coeff_hbm, tab_v)
        pltpu.sync_copy(slope_hbm, slp_v)

        def body(x_vmem, base_vmem, o_vmem):
            lo = base_vmem[0, :]
            bw_off = lo + jnp.float32(HALF)
            hi = lo + jnp.float32(2 * HALF - 1)

            @plsc.parallel_loop(0, BH * BW, step=LANES, unroll=8)
            def _(k):
                s = k // BW
                l = k % BW
                xv = x_vmem[0, 0, s, pl.ds(l, LANES)]
                wv = xv * INV_GRID + bw_off
                z = jnp.minimum(jnp.maximum(wv, lo), hi)
                i = z.astype(jnp.int32)
                frac = wv - i.astype(jnp.float32)
                idx = jax.lax.bitcast_convert_type(
                    jnp.minimum(
                        jax.lax.bitcast_convert_type(i, jnp.uint32),
                        jnp.uint32(TAB_N - 1),
                    ),
                    jnp.int32,
                )
                c0 = plsc.load_gather(tab_v, [idx])
                s0 = plsc.load_gather(slp_v, [idx])
                o_vmem[0, 0, s, pl.ds(l, LANES)] = c0 + frac * s0

        pltpu.emit_pipeline(
            body,
            grid=(rows * hb * wb,),
            in_specs=[
                pl.BlockSpec(
                    (1, 1, BH, BW),
                    lambda j: (
                        j // (NUM_ACT * hb * wb),
                        (j // (hb * wb)) % NUM_ACT,
                        (j % (hb * wb)) // wb,
                        j % wb,
                    ),
                ),
                pl.BlockSpec((1, LANES), lambda j: (j // (hb * wb), 0)),
            ],
            out_specs=[
                pl.BlockSpec(
                    (1, 1, BH, BW),
                    lambda j: (
                        j // (NUM_ACT * hb * wb),
                        (j // (hb * wb)) % NUM_ACT,
                        (j % (hb * wb)) // wb,
                        j % wb,
                    ),
                )
            ],
            core_axis_name=("core", "subcore"),
            dimension_semantics=(pltpu.PARALLEL,),
        )(x_hbm, base_hbm, o_hbm)

    out = spline_kernel(x, base_arr, coefficients_vect, slope_vect)
    return out
